# Initial kernel scaffold; baseline (speedup 1.0000x reference)
import jax
import jax.numpy as jnp
from jax.experimental import pallas as pl


def kernel(g_tok_table, e_tok_table, g_pos_table, e_pos_table, g_text_tokens, e_text_tokens):
    raise NotImplementedError("write your pallas kernel here")


# speedup vs baseline: 3.5774x; 3.5774x over previous
import functools

import jax
import jax.numpy as jnp
from jax import lax
from jax.experimental import pallas as pl
from jax.experimental.pallas import tpu as pltpu
from jax.experimental.pallas import tpu_sc as plsc

BATCH = 4096
SEQ = 200
EMBED_DIM = 64
ROWS = BATCH * SEQ
NUM_CORES = 2
NUM_SUBCORES = 16
NW = NUM_CORES * NUM_SUBCORES
RPW = ROWS // NW
CHUNK = 2 * SEQ
NCHUNK = RPW // CHUNK
SUB = 80
NSUB = CHUNK // SUB
LANES = 16
CPR = EMBED_DIM // LANES


def _body(g_tab, e_tab, g_pos, e_pos, g_idx, e_idx,
          g_out, e_out, pos_g_v, pos_e_v, idx_v, rows_v, sem):
    wid = lax.axis_index("s") * NUM_CORES + lax.axis_index("c")
    base = wid * RPW

    pltpu.sync_copy(g_pos, pos_g_v)
    pltpu.sync_copy(e_pos, pos_e_v)

    for tab, idx, out, pos_v in ((g_tab, g_idx, g_out, pos_g_v),
                                 (e_tab, e_idx, e_out, pos_e_v)):
        def chunk_body(c, _, tab=tab, idx=idx, out=out, pos_v=pos_v):
            off = base + c * CHUNK
            pltpu.sync_copy(idx.at[pl.ds(off, CHUNK)], idx_v)
            handles = [
                pltpu.async_copy(
                    tab.at[idx_v.at[pl.ds(j * SUB, SUB)]],
                    rows_v.at[pl.ds(j * SUB, SUB)],
                    sem,
                )
                for j in range(NSUB)
            ]
            for h in handles:
                h.wait()

            def row_body(r, _):
                for s in range(CHUNK // SEQ):
                    row = s * SEQ + r
                    for cc in range(CPR):
                        sl = pl.ds(cc * LANES, LANES)
                        rows_v[row, sl] = rows_v[row, sl] + pos_v[r, sl]
                return 0

            lax.fori_loop(0, SEQ, row_body, 0)
            pltpu.sync_copy(rows_v, out.at[pl.ds(off, CHUNK)])
            return 0

        lax.fori_loop(0, NCHUNK, chunk_body, 0)


@jax.jit
def kernel(g_tok_table, e_tok_table, g_pos_table, e_pos_table,
           g_text_tokens, e_text_tokens):
    g_idx = g_text_tokens.reshape(ROWS).astype(jnp.int32)
    e_idx = e_text_tokens.reshape(ROWS).astype(jnp.int32)

    mesh = plsc.VectorSubcoreMesh(core_axis_name="c", subcore_axis_name="s")
    run = functools.partial(
        pl.kernel,
        mesh=mesh,
        compiler_params=pltpu.CompilerParams(use_tc_tiling_on_sc=False),
        out_type=(
            jax.ShapeDtypeStruct((ROWS, EMBED_DIM), jnp.float32),
            jax.ShapeDtypeStruct((ROWS, EMBED_DIM), jnp.float32),
        ),
        scratch_types=[
            pltpu.VMEM((SEQ, EMBED_DIM), jnp.float32),
            pltpu.VMEM((SEQ, EMBED_DIM), jnp.float32),
            pltpu.VMEM((CHUNK,), jnp.int32),
            pltpu.VMEM((CHUNK, EMBED_DIM), jnp.float32),
            pltpu.SemaphoreType.DMA,
        ],
    )(_body)
    g_out, e_out = run(g_tok_table, e_tok_table, g_pos_table, e_pos_table,
                       g_idx, e_idx)
    return (g_out.reshape(BATCH, SEQ, EMBED_DIM),
            e_out.reshape(BATCH, SEQ, EMBED_DIM))


# speedup vs baseline: 4.3525x; 1.2167x over previous
import functools

import jax
import jax.numpy as jnp
from jax import lax
from jax.experimental import pallas as pl
from jax.experimental.pallas import tpu as pltpu
from jax.experimental.pallas import tpu_sc as plsc

BATCH = 4096
SEQ = 200
EMBED_DIM = 64
ROWS = BATCH * SEQ
NUM_CORES = 2
NUM_SUBCORES = 16
NW = NUM_CORES * NUM_SUBCORES
RPW = ROWS // NW
SPC = 2
CHUNK = SPC * SEQ
NCHUNK = RPW // CHUNK
NBUF = 4
NITER = NCHUNK // NBUF
SUB = 80
NSUB = CHUNK // SUB
LANES = 16
CPR = EMBED_DIM // LANES


def _body(g_tab, e_tab, g_pos, e_pos, g_idx, e_idx, g_out, e_out,
          pos_v,
          idx0_v, idx1_v, idx2_v, idx3_v,
          rows0_v, rows1_v, rows2_v, rows3_v,
          gsem0, gsem1, gsem2, gsem3,
          osem0, osem1, osem2, osem3,
          isem0, isem1, isem2, isem3):
    wid = lax.axis_index("s") * NUM_CORES + lax.axis_index("c")
    base = wid * RPW
    idx_vs = (idx0_v, idx1_v, idx2_v, idx3_v)
    rows_vs = (rows0_v, rows1_v, rows2_v, rows3_v)
    gsems = (gsem0, gsem1, gsem2, gsem3)
    osems = (osem0, osem1, osem2, osem3)
    isems = (isem0, isem1, isem2, isem3)

    for tab, idx, out, pos in ((g_tab, g_idx, g_out, g_pos),
                               (e_tab, e_idx, e_out, e_pos)):
        pltpu.sync_copy(pos, pos_v)

        def start_idx(c, b, idx=idx):
            pltpu.async_copy(
                idx.at[pl.ds(base + c * CHUNK, CHUNK)], idx_vs[b], isems[b])

        def wait_idx(c, b, idx=idx):
            pltpu.make_async_copy(
                idx.at[pl.ds(base + c * CHUNK, CHUNK)], idx_vs[b],
                isems[b]).wait()

        def start_gathers(b, tab=tab):
            for j in range(NSUB):
                pltpu.async_copy(
                    tab.at[idx_vs[b].at[pl.ds(j * SUB, SUB)]],
                    rows_vs[b].at[pl.ds(j * SUB, SUB)],
                    gsems[b],
                )

        def wait_gathers(c, b, out=out):
            pltpu.make_async_copy(
                out.at[pl.ds(base + c * CHUNK, CHUNK)], rows_vs[b],
                gsems[b]).wait()

        def start_out(c, b, out=out):
            pltpu.async_copy(
                rows_vs[b], out.at[pl.ds(base + c * CHUNK, CHUNK)], osems[b])

        def wait_out(c, b, out=out):
            pltpu.make_async_copy(
                rows_vs[b], out.at[pl.ds(base + c * CHUNK, CHUNK)],
                osems[b]).wait()

        def add_pos(b):
            rows_v = rows_vs[b]

            def row_body(r, _):
                for cc in range(CPR):
                    sl = pl.ds(cc * LANES, LANES)
                    p = pos_v[r, sl]
                    for s in range(SPC):
                        row = s * SEQ + r
                        rows_v[row, sl] = rows_v[row, sl] + p
                return 0

            lax.fori_loop(0, SEQ, row_body, 0)

        for b in range(NBUF):
            start_idx(b, b)
        for b in range(NBUF):
            wait_idx(b, b)
            start_gathers(b)

        def turn_body(i, _):
            for b in range(NBUF):
                c = NBUF * i + b
                wait_gathers(c, b)

                @pl.when(i < NITER - 1)
                def _(c=c, b=b):
                    start_idx(c + NBUF, b)

                add_pos(b)
                start_out(c, b)

            @pl.when(i < NITER - 1)
            def _():
                for b in range(NBUF):
                    c = NBUF * i + b
                    wait_out(c, b)
                    wait_idx(c + NBUF, b)
                    start_gathers(b)

            return 0

        lax.fori_loop(0, NITER, turn_body, 0)
        for b in range(NBUF):
            wait_out(NCHUNK - NBUF + b, b)


@jax.jit
def kernel(g_tok_table, e_tok_table, g_pos_table, e_pos_table,
           g_text_tokens, e_text_tokens):
    g_idx = g_text_tokens.reshape(ROWS).astype(jnp.int32)
    e_idx = e_text_tokens.reshape(ROWS).astype(jnp.int32)

    mesh = plsc.VectorSubcoreMesh(core_axis_name="c", subcore_axis_name="s")
    run = functools.partial(
        pl.kernel,
        mesh=mesh,
        compiler_params=pltpu.CompilerParams(use_tc_tiling_on_sc=False),
        out_type=(
            jax.ShapeDtypeStruct((ROWS, EMBED_DIM), jnp.float32),
            jax.ShapeDtypeStruct((ROWS, EMBED_DIM), jnp.float32),
        ),
        scratch_types=[
            pltpu.VMEM((SEQ, EMBED_DIM), jnp.float32),
        ] + [pltpu.VMEM((CHUNK,), jnp.int32)] * NBUF
          + [pltpu.VMEM((CHUNK, EMBED_DIM), jnp.float32)] * NBUF
          + [pltpu.SemaphoreType.DMA] * (3 * NBUF),
    )(_body)
    g_out, e_out = run(g_tok_table, e_tok_table, g_pos_table, e_pos_table,
                       g_idx, e_idx)
    return (g_out.reshape(BATCH, SEQ, EMBED_DIM),
            e_out.reshape(BATCH, SEQ, EMBED_DIM))
